# baseline (speedup 1.0000x reference)
import jax
import jax.numpy as jnp
from jax.experimental import pallas as pl

N = 100000
D = 128
H = 128
O = 128
T = 4
K = 8
R = 2000
NEG = -1e5


def _body(nt_ref, x_ref, w1_ref, p1_ref, w2_ref, b2_ref, o_ref):
    xb = x_ref[...].astype(jnp.bfloat16)
    t = nt_ref[0]
    oh = jnp.where(
        t == jax.lax.broadcasted_iota(jnp.int32, (R, K), 1), 1.0, 0.0
    ).astype(jnp.bfloat16)
    h = jnp.dot(xb, w1_ref[...], preferred_element_type=jnp.float32)
    h = h + jnp.dot(oh, p1_ref[...], preferred_element_type=jnp.float32)
    h = jnp.maximum(h, 0.0).astype(jnp.bfloat16)
    o = jnp.dot(h, w2_ref[...], preferred_element_type=jnp.float32)
    o = o + jnp.dot(oh, b2_ref[...], preferred_element_type=jnp.float32)
    o_ref[...] = o


def kernel(x, node_types, W1, b1, W2, b2):
    nb = N // R
    w1cat = jnp.concatenate(
        [jnp.transpose(W1[t]) for t in range(T)], axis=1
    )
    w2stk = jnp.transpose(W2, (0, 2, 1)).reshape(T * H, O)
    grp = jnp.arange(T * H) // H
    p1 = jnp.where(
        grp[None, :] == jnp.arange(K)[:, None],
        jnp.broadcast_to(b1.reshape(1, T * H), (K, T * H)),
        NEG,
    )
    b2a = jnp.concatenate([b2, jnp.zeros((K - T, O), b2.dtype)], axis=0)
    nt3 = node_types.reshape(nb, R, 1)
    grid_spec = pl.GridSpec(
        grid=(nb,),
        in_specs=[
            pl.BlockSpec((1, R, 1), lambda i: (i, 0, 0)),
            pl.BlockSpec((R, D), lambda i: (i, 0)),
            pl.BlockSpec((D, T * H), lambda i: (0, 0)),
            pl.BlockSpec((K, T * H), lambda i: (0, 0)),
            pl.BlockSpec((T * H, O), lambda i: (0, 0)),
            pl.BlockSpec((K, O), lambda i: (0, 0)),
        ],
        out_specs=pl.BlockSpec((R, O), lambda i: (i, 0)),
    )
    return pl.pallas_call(
        _body,
        grid_spec=grid_spec,
        out_shape=jax.ShapeDtypeStruct((N, O), jnp.float32),
    )(
        nt3,
        x,
        w1cat.astype(jnp.bfloat16),
        p1.astype(jnp.bfloat16),
        w2stk.astype(jnp.bfloat16),
        b2a,
    )


# speedup vs baseline: 1.2483x; 1.2483x over previous
import jax
import jax.numpy as jnp
from jax.experimental import pallas as pl

N = 100000
D = 128
H = 128
O = 128
T = 4
R = 4000


def _body(nt_ref, x_ref, w1_ref, b1_ref, w2_ref, b2_ref, o_ref):
    xb = x_ref[...].astype(jnp.bfloat16)
    h = jnp.dot(xb, w1_ref[...], preferred_element_type=jnp.float32)
    h = jnp.maximum(h + b1_ref[...], 0.0)
    t = nt_ref[0]
    grp = jax.lax.broadcasted_iota(jnp.int32, (1, T * H), 1) // H
    h = jnp.where(grp == t, h, 0.0).astype(jnp.bfloat16)
    o = jnp.dot(h, w2_ref[...], preferred_element_type=jnp.float32)
    b2 = b2_ref[...]
    for tt in range(T):
        o = o + jnp.where(t == tt, b2[tt][None, :], 0.0)
    o_ref[...] = o


def kernel(x, node_types, W1, b1, W2, b2):
    nb = N // R
    w1cat = jnp.concatenate([jnp.transpose(W1[t]) for t in range(T)], axis=1)
    w2stk = jnp.transpose(W2, (0, 2, 1)).reshape(T * H, O)
    b1cat = b1.reshape(1, T * H)
    nt3 = node_types.reshape(nb, R, 1)
    grid_spec = pl.GridSpec(
        grid=(nb,),
        in_specs=[
            pl.BlockSpec((1, R, 1), lambda i: (i, 0, 0)),
            pl.BlockSpec((R, D), lambda i: (i, 0)),
            pl.BlockSpec((D, T * H), lambda i: (0, 0)),
            pl.BlockSpec((1, T * H), lambda i: (0, 0)),
            pl.BlockSpec((T * H, O), lambda i: (0, 0)),
            pl.BlockSpec((T, O), lambda i: (0, 0)),
        ],
        out_specs=pl.BlockSpec((R, O), lambda i: (i, 0)),
    )
    return pl.pallas_call(
        _body,
        grid_spec=grid_spec,
        out_shape=jax.ShapeDtypeStruct((N, O), jnp.float32),
    )(nt3, x, w1cat.astype(jnp.bfloat16), b1cat, w2stk.astype(jnp.bfloat16), b2)


# speedup vs baseline: 1.3029x; 1.0438x over previous
import jax
import jax.numpy as jnp
from jax.experimental import pallas as pl

N = 100000
D = 128
H = 128
O = 128
T = 4
K = 8
R = 4000
NEG = -1e5


def _body(nt_ref, x_ref, w1_ref, w2_ref, b2_ref, o_ref):
    t = nt_ref[0]
    oh = jnp.where(
        t == jax.lax.broadcasted_iota(jnp.int32, (R, K), 1), 1.0, 0.0
    ).astype(jnp.bfloat16)
    xb = x_ref[...].astype(jnp.bfloat16)
    xcat = jnp.concatenate([xb, oh], axis=1)
    h = jnp.dot(xcat, w1_ref[...], preferred_element_type=jnp.float32)
    hb = jnp.maximum(h, 0.0).astype(jnp.bfloat16)
    o = jnp.dot(hb, w2_ref[...], preferred_element_type=jnp.float32)
    o = o + jnp.dot(oh, b2_ref[...], preferred_element_type=jnp.float32)
    o_ref[...] = o


def kernel(x, node_types, W1, b1, W2, b2):
    nb = N // R
    w1cat = jnp.concatenate([jnp.transpose(W1[t]) for t in range(T)], axis=1)
    grp = jnp.arange(T * H) // H
    p1 = jnp.where(
        grp[None, :] == jnp.arange(K)[:, None],
        jnp.broadcast_to(b1.reshape(1, T * H), (K, T * H)),
        NEG,
    )
    w1aug = jnp.concatenate([w1cat, p1], axis=0)
    w2stk = jnp.transpose(W2, (0, 2, 1)).reshape(T * H, O)
    b2a = jnp.concatenate([b2, jnp.zeros((K - T, O), b2.dtype)], axis=0)
    nt3 = node_types.reshape(nb, R, 1)
    grid_spec = pl.GridSpec(
        grid=(nb,),
        in_specs=[
            pl.BlockSpec((1, R, 1), lambda i: (i, 0, 0)),
            pl.BlockSpec((R, D), lambda i: (i, 0)),
            pl.BlockSpec((D + K, T * H), lambda i: (0, 0)),
            pl.BlockSpec((T * H, O), lambda i: (0, 0)),
            pl.BlockSpec((K, O), lambda i: (0, 0)),
        ],
        out_specs=pl.BlockSpec((R, O), lambda i: (i, 0)),
    )
    return pl.pallas_call(
        _body,
        grid_spec=grid_spec,
        out_shape=jax.ShapeDtypeStruct((N, O), jnp.float32),
    )(
        nt3,
        x,
        w1aug.astype(jnp.bfloat16),
        w2stk.astype(jnp.bfloat16),
        b2a,
    )


# speedup vs baseline: 2.1499x; 1.6501x over previous
import jax
import jax.numpy as jnp
from jax.experimental import pallas as pl

N = 100000
D = 128
H = 128
O = 128
T = 4
K = 8
R = 4000
NEG = -1e5


def _body(oht_ref, x_ref, w1_ref, w2_ref, b2_ref, o_ref):
    oh = jnp.transpose(oht_ref[0])
    xb = x_ref[...].astype(jnp.bfloat16)
    xcat = jnp.concatenate([xb, oh], axis=1)
    h = jnp.dot(xcat, w1_ref[...], preferred_element_type=jnp.float32)
    hb = jnp.maximum(h, 0.0).astype(jnp.bfloat16)
    o = jnp.dot(hb, w2_ref[...], preferred_element_type=jnp.float32)
    o = o + jnp.dot(oh, b2_ref[...], preferred_element_type=jnp.float32)
    o_ref[...] = o


def kernel(x, node_types, W1, b1, W2, b2):
    nb = N // R
    nt_blocks = node_types.reshape(nb, R)
    oht = (
        jnp.arange(K, dtype=node_types.dtype)[None, :, None]
        == nt_blocks[:, None, :]
    ).astype(jnp.bfloat16)
    w1cat = jnp.concatenate([jnp.transpose(W1[t]) for t in range(T)], axis=1)
    grp = jnp.arange(T * H) // H
    p1 = jnp.where(
        grp[None, :] == jnp.arange(K)[:, None],
        jnp.broadcast_to(b1.reshape(1, T * H), (K, T * H)),
        NEG,
    )
    w1aug = jnp.concatenate([w1cat, p1], axis=0)
    w2stk = jnp.transpose(W2, (0, 2, 1)).reshape(T * H, O)
    b2a = jnp.concatenate([b2, jnp.zeros((K - T, O), b2.dtype)], axis=0)
    grid_spec = pl.GridSpec(
        grid=(nb,),
        in_specs=[
            pl.BlockSpec((1, K, R), lambda i: (i, 0, 0)),
            pl.BlockSpec((R, D), lambda i: (i, 0)),
            pl.BlockSpec((D + K, T * H), lambda i: (0, 0)),
            pl.BlockSpec((T * H, O), lambda i: (0, 0)),
            pl.BlockSpec((K, O), lambda i: (0, 0)),
        ],
        out_specs=pl.BlockSpec((R, O), lambda i: (i, 0)),
    )
    return pl.pallas_call(
        _body,
        grid_spec=grid_spec,
        out_shape=jax.ShapeDtypeStruct((N, O), jnp.float32),
    )(
        oht,
        x,
        w1aug.astype(jnp.bfloat16),
        w2stk.astype(jnp.bfloat16),
        b2a,
    )
